# Initial kernel scaffold; baseline (speedup 1.0000x reference)
import jax
import jax.numpy as jnp
from jax.experimental import pallas as pl


def kernel(node_feats, edge_feats, edge_index, W1, b1, W2, b2, ln_gamma, ln_beta):
    raise NotImplementedError("write your pallas kernel here")


# speedup vs baseline: 2.8100x; 2.8100x over previous
import functools
import math

import jax
import jax.numpy as jnp
from jax import lax
from jax.experimental import pallas as pl
from jax.experimental.pallas import tpu as pltpu
from jax.experimental.pallas import tpu_sc as plsc

_L = 16


def _sc_message_pass(node_feats, edge_feats, src, dst):
    n_nodes, d = node_feats.shape
    n_edges = src.shape[0]
    info = plsc.get_sparse_core_info()
    nc, ns = info.num_cores, info.num_subcores

    half = n_nodes // nc
    zrows = 112
    per_tile_zero = -(-(half + 1) // (ns * zrows)) * zrows
    acc_rows = per_tile_zero * ns
    epc = n_edges // ns
    C = 80
    n_chunks = epc // C
    assert n_chunks * C == epc and half * nc == n_nodes

    mesh = plsc.VectorSubcoreMesh(core_axis_name="c", subcore_axis_name="s")

    @functools.partial(
        pl.kernel,
        out_type=jax.ShapeDtypeStruct((n_nodes, d), jnp.float32),
        mesh=mesh,
        scratch_types=[
            pltpu.VMEM((C,), jnp.int32),
            pltpu.VMEM((C,), jnp.int32),
            pltpu.VMEM((C,), jnp.int32),
            pltpu.VMEM((C, d), jnp.float32),
            pltpu.VMEM((C, d), jnp.float32),
            pltpu.VMEM((zrows, d), jnp.float32),
            pltpu.VMEM_SHARED((acc_rows, d), jnp.float32),
            pltpu.SemaphoreType.DMA,
            pltpu.SemaphoreType.DMA,
        ],
        compiler_params=pltpu.CompilerParams(use_tc_tiling_on_sc=False),
    )
    def mp(node_hbm, edge_hbm, src_hbm, dst_hbm, out_hbm,
           src_v, dst_v, idx_v, gbuf, ebuf, zbuf, acc, gsem, esem):
        cid = lax.axis_index("c")
        sid = lax.axis_index("s")
        base_node = cid * half

        zero = jnp.zeros((_L,), jnp.float32)
        for r in range(zrows):
            for h in range(d // _L):
                zbuf[r, pl.ds(h * _L, _L)] = zero

        def zero_body(k, carry):
            pltpu.sync_copy(zbuf, acc.at[pl.ds(sid * per_tile_zero + k * zrows, zrows)])
            return carry
        lax.fori_loop(0, per_tile_zero // zrows, zero_body, 0)
        plsc.subcore_barrier()

        tile_base = sid * epc

        def chunk_body(j, carry):
            b = pl.multiple_of(tile_base + j * C, 8)
            pltpu.sync_copy(src_hbm.at[pl.ds(b, C)], src_v)
            pltpu.sync_copy(dst_hbm.at[pl.ds(b, C)], dst_v)
            gcp = pltpu.async_copy(node_hbm.at[src_v], gbuf, gsem)
            ecp = pltpu.async_copy(edge_hbm.at[pl.ds(b, C)], ebuf, esem)
            for k in range(C // _L):
                v = dst_v[pl.ds(k * _L, _L)]
                loc = v - base_node
                ok = (loc >= 0) & (loc < half)
                idx_v[pl.ds(k * _L, _L)] = jnp.where(ok, loc, half)
            gcp.wait()
            ecp.wait()
            pltpu.sync_copy(gbuf, acc.at[idx_v], add=True)
            pltpu.sync_copy(ebuf, acc.at[idx_v], add=True)
            return carry
        lax.fori_loop(0, n_chunks, chunk_body, 0)
        plsc.subcore_barrier()

        w0 = -(-(half // ns) // 8) * 8
        w_last = half - (ns - 1) * w0
        start = pl.multiple_of(sid * w0, 8)
        obase = pl.multiple_of(cid * half + sid * w0, 8)

        @pl.when(sid < ns - 1)
        def _():
            pltpu.sync_copy(acc.at[pl.ds(start, w0)], out_hbm.at[pl.ds(obase, w0)])

        @pl.when(sid == ns - 1)
        def _():
            pltpu.sync_copy(acc.at[pl.ds(start, w_last)],
                            out_hbm.at[pl.ds(obase, w_last)])

    return mp(node_feats, edge_feats, src, dst)


def _epilogue_body(inv_sqrt_n, a_ref, nf_ref, w1_ref, b1_ref, w2_ref, b2_ref,
                   g_ref, bt_ref, o_ref):
    h1 = jnp.maximum(
        jnp.dot(a_ref[...], w1_ref[...], preferred_element_type=jnp.float32)
        + b1_ref[...], 0.0)
    h = jnp.dot(h1, w2_ref[...], preferred_element_type=jnp.float32) + b2_ref[...]
    mu = jnp.mean(h, axis=-1, keepdims=True)
    var = jnp.mean((h - mu) ** 2, axis=-1, keepdims=True)
    h = (h - mu) * lax.rsqrt(var + 1e-5) * g_ref[...] + bt_ref[...]
    h = jnp.maximum(h * inv_sqrt_n, 0.0)
    o_ref[...] = h + nf_ref[...]


def _tc_epilogue(agg, node_feats, W1, b1, W2, b2, ln_gamma, ln_beta):
    n, d = agg.shape
    rows = 4000
    grid = n // rows
    assert grid * rows == n
    b1r = b1.reshape(1, -1)
    b2r = b2.reshape(1, -1)
    gr = ln_gamma.reshape(1, -1)
    br = ln_beta.reshape(1, -1)
    inv = 1.0 / math.sqrt(n)
    full = lambda i: (0, 0)
    return pl.pallas_call(
        functools.partial(_epilogue_body, inv),
        grid=(grid,),
        in_specs=[
            pl.BlockSpec((rows, d), lambda i: (i, 0)),
            pl.BlockSpec((rows, d), lambda i: (i, 0)),
            pl.BlockSpec((d, 2 * d), full),
            pl.BlockSpec((1, 2 * d), full),
            pl.BlockSpec((2 * d, d), full),
            pl.BlockSpec((1, d), full),
            pl.BlockSpec((1, d), full),
            pl.BlockSpec((1, d), full),
        ],
        out_specs=pl.BlockSpec((rows, d), lambda i: (i, 0)),
        out_shape=jax.ShapeDtypeStruct((n, d), jnp.float32),
    )(agg, node_feats, W1, b1r, W2, b2r, gr, br)


def kernel(node_feats, edge_feats, edge_index, W1, b1, W2, b2, ln_gamma, ln_beta):
    src = edge_index[0]
    dst = edge_index[1]
    agg = _sc_message_pass(node_feats, edge_feats, src, dst)
    return _tc_epilogue(agg, node_feats, W1, b1, W2, b2, ln_gamma, ln_beta)


# speedup vs baseline: 3.3191x; 1.1812x over previous
import functools
import math

import jax
import jax.numpy as jnp
from jax import lax
from jax.experimental import pallas as pl
from jax.experimental.pallas import tpu as pltpu
from jax.experimental.pallas import tpu_sc as plsc

_L = 16


def _sc_message_pass(node_feats, edge_feats, src, dst):
    n_nodes, d = node_feats.shape
    n_edges = src.shape[0]
    info = plsc.get_sparse_core_info()
    nc, ns = info.num_cores, info.num_subcores

    half = n_nodes // nc
    zrows = 112
    per_tile_zero = -(-(half + 1) // (ns * zrows)) * zrows
    acc_rows = per_tile_zero * ns
    epc = n_edges // ns
    C = 80
    n_chunks = epc // C
    NSETS = 4
    assert n_chunks * C == epc and half * nc == n_nodes
    assert n_chunks % NSETS == 2

    mesh = plsc.VectorSubcoreMesh(core_axis_name="c", subcore_axis_name="s")

    @functools.partial(
        pl.kernel,
        out_type=jax.ShapeDtypeStruct((n_nodes, d), jnp.float32),
        mesh=mesh,
        scratch_types=[
            [pltpu.VMEM((C,), jnp.int32)] * NSETS,
            [pltpu.VMEM((C,), jnp.int32)] * NSETS,
            [pltpu.VMEM((C,), jnp.int32)] * NSETS,
            [pltpu.VMEM((C, d), jnp.float32)] * NSETS,
            [pltpu.VMEM((C, d), jnp.float32)] * NSETS,
            pltpu.VMEM((zrows, d), jnp.float32),
            pltpu.VMEM_SHARED((acc_rows, d), jnp.float32),
            [pltpu.SemaphoreType.DMA] * NSETS,
            [pltpu.SemaphoreType.DMA] * NSETS,
            [pltpu.SemaphoreType.DMA] * NSETS,
            [pltpu.SemaphoreType.DMA] * NSETS,
        ],
        compiler_params=pltpu.CompilerParams(use_tc_tiling_on_sc=False),
    )
    def mp(node_hbm, edge_hbm, src_hbm, dst_hbm, out_hbm,
           srcS, dstS, idxS, gS, eS, zbuf, acc, isem, gsem, esem, ssem):
        cid = lax.axis_index("c")
        sid = lax.axis_index("s")
        base_node = cid * half

        zero = jnp.zeros((_L,), jnp.float32)
        for r in range(zrows):
            for h in range(d // _L):
                zbuf[r, pl.ds(h * _L, _L)] = zero

        def zero_body(k, carry):
            pltpu.sync_copy(zbuf, acc.at[pl.ds(sid * per_tile_zero + k * zrows, zrows)])
            return carry
        lax.fori_loop(0, per_tile_zero // zrows, zero_body, 0)
        plsc.subcore_barrier()

        tile_base = sid * epc

        def ebase(j):
            return pl.multiple_of(tile_base + j * C, 8)

        def L(j, s):
            b = ebase(j)
            pltpu.async_copy(src_hbm.at[pl.ds(b, C)], srcS[s], isem[s])
            pltpu.async_copy(dst_hbm.at[pl.ds(b, C)], dstS[s], isem[s])

        def Lw(s):
            pltpu.make_async_copy(src_hbm.at[pl.ds(0, C)], srcS[s], isem[s]).wait()
            pltpu.make_async_copy(dst_hbm.at[pl.ds(0, C)], dstS[s], isem[s]).wait()

        def G(j, s):
            pltpu.async_copy(node_hbm.at[srcS[s]], gS[s], gsem[s])
            pltpu.async_copy(edge_hbm.at[pl.ds(ebase(j), C)], eS[s], esem[s])

        def Gw(s):
            pltpu.make_async_copy(node_hbm.at[srcS[s]], gS[s], gsem[s]).wait()
            pltpu.make_async_copy(edge_hbm.at[pl.ds(0, C)], eS[s], esem[s]).wait()

        def comp(s):
            for k in range(C // _L):
                v = dstS[s][pl.ds(k * _L, _L)]
                loc = v - base_node
                ok = (loc >= 0) & (loc < half)
                idxS[s][pl.ds(k * _L, _L)] = jnp.where(ok, loc, half)

        def S(s):
            pltpu.async_copy(gS[s], acc.at[idxS[s]], ssem[s], add=True)
            pltpu.async_copy(eS[s], acc.at[idxS[s]], ssem[s], add=True)

        def Sw(s):
            pltpu.make_async_copy(gS[s], acc.at[idxS[s]], ssem[s]).wait()
            pltpu.make_async_copy(eS[s], acc.at[idxS[s]], ssem[s]).wait()

        for s in range(NSETS):
            L(s, s)
        for s in (0, 1):
            Lw(s); comp(s); G(s, s)
        for j in (0, 1):
            s, s2 = j % NSETS, (j + 2) % NSETS
            Gw(s); L(j + 4, s); S(s)
            Lw(s2); comp(s2); G(j + 2, s2)
        for j in (2, 3):
            s, s2 = j % NSETS, (j + 2) % NSETS
            Gw(s); L(j + 4, s); S(s); Sw(s2)
            Lw(s2); comp(s2); G(j + 2, s2)

        def body(i, carry):
            for u in range(NSETS):
                s, s2 = u, (u + 2) % NSETS
                j = i * NSETS + u
                Gw(s)

                @pl.when(j + 4 < n_chunks)
                def _():
                    L(j + 4, s)
                S(s); Sw(s2)
                Lw(s2); comp(s2); G(j + 2, s2)
            return carry
        lax.fori_loop(1, n_chunks // NSETS, body, 0)

        for j in (n_chunks - 2, n_chunks - 1):
            s, s2 = j % NSETS, (j + 2) % NSETS
            Gw(s); S(s); Sw(s2)
        Sw((n_chunks - 2) % NSETS)
        Sw((n_chunks - 1) % NSETS)
        plsc.subcore_barrier()

        w0 = -(-(half // ns) // 8) * 8
        w_last = half - (ns - 1) * w0
        start = pl.multiple_of(sid * w0, 8)
        obase = pl.multiple_of(cid * half + sid * w0, 8)

        @pl.when(sid < ns - 1)
        def _():
            pltpu.sync_copy(acc.at[pl.ds(start, w0)], out_hbm.at[pl.ds(obase, w0)])

        @pl.when(sid == ns - 1)
        def _():
            pltpu.sync_copy(acc.at[pl.ds(start, w_last)],
                            out_hbm.at[pl.ds(obase, w_last)])

    return mp(node_feats, edge_feats, src, dst)


def _epilogue_body(inv_sqrt_n, a_ref, nf_ref, w1_ref, b1_ref, w2_ref, b2_ref,
                   g_ref, bt_ref, o_ref):
    h1 = jnp.maximum(
        jnp.dot(a_ref[...], w1_ref[...], preferred_element_type=jnp.float32)
        + b1_ref[...], 0.0)
    h = jnp.dot(h1, w2_ref[...], preferred_element_type=jnp.float32) + b2_ref[...]
    mu = jnp.mean(h, axis=-1, keepdims=True)
    var = jnp.mean((h - mu) ** 2, axis=-1, keepdims=True)
    h = (h - mu) * lax.rsqrt(var + 1e-5) * g_ref[...] + bt_ref[...]
    h = jnp.maximum(h * inv_sqrt_n, 0.0)
    o_ref[...] = h + nf_ref[...]


def _tc_epilogue(agg, node_feats, W1, b1, W2, b2, ln_gamma, ln_beta):
    n, d = agg.shape
    rows = 4000
    grid = n // rows
    assert grid * rows == n
    b1r = b1.reshape(1, -1)
    b2r = b2.reshape(1, -1)
    gr = ln_gamma.reshape(1, -1)
    br = ln_beta.reshape(1, -1)
    inv = 1.0 / math.sqrt(n)
    full = lambda i: (0, 0)
    return pl.pallas_call(
        functools.partial(_epilogue_body, inv),
        grid=(grid,),
        in_specs=[
            pl.BlockSpec((rows, d), lambda i: (i, 0)),
            pl.BlockSpec((rows, d), lambda i: (i, 0)),
            pl.BlockSpec((d, 2 * d), full),
            pl.BlockSpec((1, 2 * d), full),
            pl.BlockSpec((2 * d, d), full),
            pl.BlockSpec((1, d), full),
            pl.BlockSpec((1, d), full),
            pl.BlockSpec((1, d), full),
        ],
        out_specs=pl.BlockSpec((rows, d), lambda i: (i, 0)),
        out_shape=jax.ShapeDtypeStruct((n, d), jnp.float32),
    )(agg, node_feats, W1, b1r, W2, b2r, gr, br)


def kernel(node_feats, edge_feats, edge_index, W1, b1, W2, b2, ln_gamma, ln_beta):
    src = edge_index[0]
    dst = edge_index[1]
    agg = _sc_message_pass(node_feats, edge_feats, src, dst)
    return _tc_epilogue(agg, node_feats, W1, b1, W2, b2, ln_gamma, ln_beta)


# speedup vs baseline: 6.3470x; 1.9123x over previous
import functools
import math

import jax
import jax.numpy as jnp
from jax import lax
from jax.experimental import pallas as pl
from jax.experimental.pallas import tpu as pltpu
from jax.experimental.pallas import tpu_sc as plsc

_L = 16


def _sc_message_pass(node_feats, edge_feats, src, dst):
    n_nodes, d = node_feats.shape
    n_edges = src.shape[0]
    info = plsc.get_sparse_core_info()
    nc, ns = info.num_cores, info.num_subcores

    half = n_nodes // nc
    zrows = 112
    per_tile_zero = -(-(half + 1 + ns * _L) // (ns * zrows)) * zrows
    acc_rows = per_tile_zero * ns
    epc = n_edges // ns
    C = 80
    n_chunks = epc // C
    NSETS = 4
    assert n_chunks * C == epc and half * nc == n_nodes
    assert n_chunks % NSETS == 2

    mesh = plsc.VectorSubcoreMesh(core_axis_name="c", subcore_axis_name="s")

    @functools.partial(
        pl.kernel,
        out_type=jax.ShapeDtypeStruct((n_nodes, d), jnp.float32),
        mesh=mesh,
        scratch_types=[
            [pltpu.VMEM((C,), jnp.int32)] * NSETS,
            [pltpu.VMEM((C,), jnp.int32)] * NSETS,
            [pltpu.VMEM((C,), jnp.int32)] * NSETS,
            [pltpu.VMEM((C, d), jnp.float32)] * NSETS,
            [pltpu.VMEM((C, d), jnp.float32)] * NSETS,
            pltpu.VMEM((zrows, d), jnp.float32),
            pltpu.VMEM_SHARED((acc_rows, d), jnp.float32),
            [pltpu.SemaphoreType.DMA] * NSETS,
            [pltpu.SemaphoreType.DMA] * NSETS,
            [pltpu.SemaphoreType.DMA] * NSETS,
            [pltpu.SemaphoreType.DMA] * NSETS,
        ],
        compiler_params=pltpu.CompilerParams(use_tc_tiling_on_sc=False),
    )
    def mp(node_hbm, edge_hbm, src_hbm, dst_hbm, out_hbm,
           srcS, dstS, idxS, gS, eS, zbuf, acc, isem, gsem, esem, ssem):
        cid = lax.axis_index("c")
        sid = lax.axis_index("s")
        base_node = cid * half

        zero = jnp.zeros((_L,), jnp.float32)
        for r in range(zrows):
            for h in range(d // _L):
                zbuf[r, pl.ds(h * _L, _L)] = zero

        def zero_body(k, carry):
            pltpu.sync_copy(zbuf, acc.at[pl.ds(sid * per_tile_zero + k * zrows, zrows)])
            return carry
        lax.fori_loop(0, per_tile_zero // zrows, zero_body, 0)
        plsc.subcore_barrier()

        tile_base = sid * epc

        def ebase(j):
            return pl.multiple_of(tile_base + j * C, 8)

        def L(j, s):
            b = ebase(j)
            pltpu.async_copy(src_hbm.at[pl.ds(b, C)], srcS[s], isem[s])
            pltpu.async_copy(dst_hbm.at[pl.ds(b, C)], dstS[s], isem[s])

        def Lw(s):
            pltpu.make_async_copy(src_hbm.at[pl.ds(0, C)], srcS[s], isem[s]).wait()
            pltpu.make_async_copy(dst_hbm.at[pl.ds(0, C)], dstS[s], isem[s]).wait()

        def G(j, s):
            pltpu.async_copy(node_hbm.at[srcS[s]], gS[s], gsem[s])
            pltpu.async_copy(edge_hbm.at[pl.ds(ebase(j), C)], eS[s], esem[s])

        def Gw(s):
            pltpu.make_async_copy(node_hbm.at[srcS[s]], gS[s], gsem[s]).wait()
            pltpu.make_async_copy(edge_hbm.at[pl.ds(0, C)], eS[s], esem[s]).wait()

        trash = half + sid * _L + lax.iota(jnp.int32, _L)

        def comp(s):
            for k in range(C // _L):
                v = dstS[s][pl.ds(k * _L, _L)]
                loc = v - base_node
                ok = (loc >= 0) & (loc < half)
                idxS[s][pl.ds(k * _L, _L)] = jnp.where(ok, loc, trash)

        def S(s):
            pltpu.async_copy(gS[s], acc.at[idxS[s]], ssem[s], add=True)
            pltpu.async_copy(eS[s], acc.at[idxS[s]], ssem[s], add=True)

        def Sw(s):
            pltpu.make_async_copy(gS[s], acc.at[idxS[s]], ssem[s]).wait()
            pltpu.make_async_copy(eS[s], acc.at[idxS[s]], ssem[s]).wait()

        for s in range(NSETS):
            L(s, s)
        for s in (0, 1):
            Lw(s); comp(s); G(s, s)
        for j in (0, 1):
            s, s2 = j % NSETS, (j + 2) % NSETS
            Gw(s); L(j + 4, s); S(s)
            Lw(s2); comp(s2); G(j + 2, s2)
        for j in (2, 3):
            s, s2 = j % NSETS, (j + 2) % NSETS
            Gw(s); L(j + 4, s); S(s); Sw(s2)
            Lw(s2); comp(s2); G(j + 2, s2)

        def body(i, carry):
            for u in range(NSETS):
                s, s2 = u, (u + 2) % NSETS
                j = i * NSETS + u
                Gw(s)

                @pl.when(j + 4 < n_chunks)
                def _():
                    L(j + 4, s)
                S(s); Sw(s2)
                Lw(s2); comp(s2); G(j + 2, s2)
            return carry
        lax.fori_loop(1, n_chunks // NSETS, body, 0)

        for j in (n_chunks - 2, n_chunks - 1):
            s, s2 = j % NSETS, (j + 2) % NSETS
            Gw(s); S(s); Sw(s2)
        Sw((n_chunks - 2) % NSETS)
        Sw((n_chunks - 1) % NSETS)
        plsc.subcore_barrier()

        w0 = -(-(half // ns) // 8) * 8
        w_last = half - (ns - 1) * w0
        start = pl.multiple_of(sid * w0, 8)
        obase = pl.multiple_of(cid * half + sid * w0, 8)

        @pl.when(sid < ns - 1)
        def _():
            pltpu.sync_copy(acc.at[pl.ds(start, w0)], out_hbm.at[pl.ds(obase, w0)])

        @pl.when(sid == ns - 1)
        def _():
            pltpu.sync_copy(acc.at[pl.ds(start, w_last)],
                            out_hbm.at[pl.ds(obase, w_last)])

    return mp(node_feats, edge_feats, src, dst)


def _epilogue_body(inv_sqrt_n, a_ref, nf_ref, w1_ref, b1_ref, w2_ref, b2_ref,
                   g_ref, bt_ref, o_ref):
    h1 = jnp.maximum(
        jnp.dot(a_ref[...], w1_ref[...], preferred_element_type=jnp.float32)
        + b1_ref[...], 0.0)
    h = jnp.dot(h1, w2_ref[...], preferred_element_type=jnp.float32) + b2_ref[...]
    mu = jnp.mean(h, axis=-1, keepdims=True)
    var = jnp.mean((h - mu) ** 2, axis=-1, keepdims=True)
    h = (h - mu) * lax.rsqrt(var + 1e-5) * g_ref[...] + bt_ref[...]
    h = jnp.maximum(h * inv_sqrt_n, 0.0)
    o_ref[...] = h + nf_ref[...]


def _tc_epilogue(agg, node_feats, W1, b1, W2, b2, ln_gamma, ln_beta):
    n, d = agg.shape
    rows = 4000
    grid = n // rows
    assert grid * rows == n
    b1r = b1.reshape(1, -1)
    b2r = b2.reshape(1, -1)
    gr = ln_gamma.reshape(1, -1)
    br = ln_beta.reshape(1, -1)
    inv = 1.0 / math.sqrt(n)
    full = lambda i: (0, 0)
    return pl.pallas_call(
        functools.partial(_epilogue_body, inv),
        grid=(grid,),
        in_specs=[
            pl.BlockSpec((rows, d), lambda i: (i, 0)),
            pl.BlockSpec((rows, d), lambda i: (i, 0)),
            pl.BlockSpec((d, 2 * d), full),
            pl.BlockSpec((1, 2 * d), full),
            pl.BlockSpec((2 * d, d), full),
            pl.BlockSpec((1, d), full),
            pl.BlockSpec((1, d), full),
            pl.BlockSpec((1, d), full),
        ],
        out_specs=pl.BlockSpec((rows, d), lambda i: (i, 0)),
        out_shape=jax.ShapeDtypeStruct((n, d), jnp.float32),
    )(agg, node_feats, W1, b1r, W2, b2r, gr, br)


def kernel(node_feats, edge_feats, edge_index, W1, b1, W2, b2, ln_gamma, ln_beta):
    src = edge_index[0]
    dst = edge_index[1]
    agg = _sc_message_pass(node_feats, edge_feats, src, dst)
    return _tc_epilogue(agg, node_feats, W1, b1, W2, b2, ln_gamma, ln_beta)
